# baseline (speedup 1.0000x reference)
import functools

import jax
import jax.numpy as jnp
from jax import lax
from jax.experimental import pallas as pl
from jax.experimental.pallas import tpu as pltpu
from jax.experimental.pallas import tpu_sc as plsc

_FILL = -10000.0
_VOCAB = 100000
_RESTRICTED = 65
_T = 2048
_HEAD_DMAS = 118
_MAIN_CHUNKS = 3093
_RING = 4


def _patch_body(hs_ref, w_ref, p1_ref, p2_ref):
    logits_t = jax.lax.dot_general(
        w_ref[...], hs_ref[0],
        dimension_numbers=(((1,), (1,)), ((), ())),
        preferred_element_type=jnp.float32)
    p1_ref[...] = jnp.full(p1_ref.shape, _FILL, dtype=jnp.float32)
    p1_ref[4:68, :] = logits_t[0:64, :]
    p2_ref[...] = jnp.full(p2_ref.shape, _FILL, dtype=jnp.float32)
    p2_ref[7:8, :] = logits_t[64:65, :]


def _head_unit(k):
    u = jnp.where(k < 12, k, jnp.where(k < 115, k + 9, k + 10))
    return u * 8


def _sc_body(p1_hbm, p2_hbm, out_hbm, buf, sem):
    wid = lax.axis_index("s") * 2 + lax.axis_index("c")

    def _fill_step(i, c):
        for r in range(32):
            buf[r, pl.ds(i * 16, 16)] = jnp.full((16,), _FILL,
                                                 dtype=jnp.float32)
        return c
    lax.fori_loop(0, _T // 16, _fill_step, 0)

    @pl.when(wid == 0)
    def _head():
        def _start(k):
            pltpu.make_async_copy(
                buf.at[pl.ds(0, 8)],
                out_hbm.at[pl.ds(_head_unit(k), 8)], sem).start()

        def _wait(k):
            pltpu.make_async_copy(
                buf.at[pl.ds(0, 8)],
                out_hbm.at[pl.ds(_head_unit(k), 8)], sem).wait()

        def _step(k, c):
            _start(k)

            @pl.when(k >= _RING)
            def _():
                _wait(k - _RING)
            return c
        lax.fori_loop(0, _HEAD_DMAS, _step, 0)
        for j in range(_RING):
            _wait(_HEAD_DMAS - _RING + j)

        pltpu.sync_copy(p1_hbm, out_hbm.at[pl.ds(96, 72)])
        pltpu.sync_copy(p2_hbm, out_hbm.at[pl.ds(992, 8)])

    @pl.when(wid > 0)
    def _main():
        w = wid - 1
        n_w = jnp.where(w <= 23, 100, 99)

        def _off(k):
            return 1024 + 32 * (w + 31 * k)

        def _start(k):
            pltpu.make_async_copy(
                buf, out_hbm.at[pl.ds(_off(k), 32)], sem).start()

        def _wait(k):
            pltpu.make_async_copy(
                buf, out_hbm.at[pl.ds(_off(k), 32)], sem).wait()

        def _step(k, c):
            _start(k)

            @pl.when(k >= _RING)
            def _():
                _wait(k - _RING)
            return c
        lax.fori_loop(0, n_w, _step, 0)

        def _drain(j, c):
            _wait(n_w - _RING + j)
            return c
        lax.fori_loop(0, _RING, _drain, 0)


def kernel(hidden_states, W):
    B, T, H = hidden_states.shape
    hs = hidden_states.astype(jnp.float32)
    w_pad = jnp.zeros((128, H), dtype=jnp.float32).at[:_RESTRICTED].set(
        W.astype(jnp.float32))

    p1, p2 = pl.pallas_call(
        _patch_body,
        in_specs=[
            pl.BlockSpec((1, T, H), lambda: (0, 0, 0)),
            pl.BlockSpec((128, H), lambda: (0, 0)),
        ],
        out_specs=[
            pl.BlockSpec((72, T), lambda: (0, 0)),
            pl.BlockSpec((8, T), lambda: (0, 0)),
        ],
        out_shape=[
            jax.ShapeDtypeStruct((72, T), jnp.float32),
            jax.ShapeDtypeStruct((8, T), jnp.float32),
        ],
    )(hs, w_pad)

    mesh = plsc.VectorSubcoreMesh(core_axis_name="c", subcore_axis_name="s")
    sc_fill = functools.partial(
        pl.kernel,
        out_type=jax.ShapeDtypeStruct((_VOCAB, T), jnp.float32),
        mesh=mesh,
        scratch_types=[
            pltpu.VMEM((32, T), jnp.float32),
            pltpu.SemaphoreType.DMA,
        ],
    )(_sc_body)
    out_t = sc_fill(p1, p2)
    return jnp.swapaxes(out_t, 0, 1)[None]


# speedup vs baseline: 1.0874x; 1.0874x over previous
import jax
import jax.numpy as jnp
from jax.experimental import pallas as pl

_FILL = -10000.0
_VOCAB = 100000
_RESTRICTED = 65
_VB = 1024


def _body(hs_ref, w_ref, out_ref):
    j = pl.program_id(0)
    out_ref[...] = jnp.full(out_ref.shape, _FILL, dtype=jnp.float32)

    @pl.when(j == 0)
    def _scatter():
        logits_t = jax.lax.dot_general(
            w_ref[...], hs_ref[0],
            dimension_numbers=(((1,), (1,)), ((), ())),
            preferred_element_type=jnp.float32)
        out_ref[100:164, :] = logits_t[0:64, :]
        out_ref[999:1000, :] = logits_t[64:65, :]


def kernel(hidden_states, W):
    B, T, H = hidden_states.shape
    hs = hidden_states.astype(jnp.float32)
    w_pad = jnp.zeros((128, H), dtype=jnp.float32).at[:_RESTRICTED].set(
        W.astype(jnp.float32))

    n_blocks = pl.cdiv(_VOCAB, _VB)
    out_t = pl.pallas_call(
        _body,
        grid=(n_blocks,),
        in_specs=[
            pl.BlockSpec((1, T, H), lambda j: (0, 0, 0)),
            pl.BlockSpec((128, H), lambda j: (0, 0)),
        ],
        out_specs=pl.BlockSpec((_VB, T), lambda j: (j, 0)),
        out_shape=jax.ShapeDtypeStruct((_VOCAB, T), jnp.float32),
    )(hs, w_pad)
    return jnp.swapaxes(out_t, 0, 1)[None]


# speedup vs baseline: 1.0940x; 1.0061x over previous
import jax
import jax.numpy as jnp
from jax import lax
from jax.experimental import pallas as pl
from jax.experimental.pallas import tpu as pltpu

_FILL = -10000.0
_VOCAB = 100000
_T = 2048
_RESTRICTED = 65
_VB = 1000
_N = _VOCAB // _VB
_RING = 4


def _copy(buf, out_ref, stripe, sem):
    return pltpu.make_async_copy(
        buf, out_ref.at[pl.ds(stripe * _VB, _VB)], sem)


def _body(hs_ref, w_ref, out_ref, fillbuf, buf0, sems, sem0):
    i = pl.program_id(0)

    @pl.when(i == 0)
    def _init_fill():
        fillbuf[...] = jnp.full((_VB, _T), _FILL, dtype=jnp.float32)

    @pl.when(i >= 1)
    def _stream():
        slot = lax.rem(i, _RING)

        @pl.when(i >= _RING + 1)
        def _reclaim():
            _copy(fillbuf, out_ref, i - _RING, sems.at[slot]).wait()

        _copy(fillbuf, out_ref, i, sems.at[slot]).start()

    @pl.when(i == 1)
    def _stripe0():
        buf0[...] = jnp.full((_VB, _T), _FILL, dtype=jnp.float32)
        logits_t = jax.lax.dot_general(
            w_ref[...], hs_ref[0],
            dimension_numbers=(((1,), (1,)), ((), ())),
            preferred_element_type=jnp.float32)
        buf0[100:164, :] = logits_t[0:64, :]
        buf0[999:1000, :] = logits_t[64:65, :]
        _copy(buf0, out_ref, 0, sem0).start()

    @pl.when(i == _N - 1)
    def _drain():
        for j in range(_RING):
            stripe = _N - _RING + j
            _copy(fillbuf, out_ref, stripe,
                  sems.at[lax.rem(stripe, _RING)]).wait()
        _copy(buf0, out_ref, 0, sem0).wait()


def kernel(hidden_states, W):
    B, T, H = hidden_states.shape
    hs = hidden_states.astype(jnp.float32)
    w_pad = jnp.zeros((128, H), dtype=jnp.float32).at[:_RESTRICTED].set(
        W.astype(jnp.float32))

    out_t = pl.pallas_call(
        _body,
        grid=(_N,),
        in_specs=[
            pl.BlockSpec((1, T, H), lambda i: (0, 0, 0)),
            pl.BlockSpec((128, H), lambda i: (0, 0)),
        ],
        out_specs=pl.BlockSpec(memory_space=pltpu.MemorySpace.HBM),
        out_shape=jax.ShapeDtypeStruct((_VOCAB, T), jnp.float32),
        scratch_shapes=[
            pltpu.VMEM((_VB, _T), jnp.float32),
            pltpu.VMEM((_VB, _T), jnp.float32),
            pltpu.SemaphoreType.DMA((_RING,)),
            pltpu.SemaphoreType.DMA,
        ],
    )(hs, w_pad)
    return jnp.swapaxes(out_t, 0, 1)[None]


# speedup vs baseline: 1.0984x; 1.0040x over previous
import jax
import jax.numpy as jnp
from jax import lax
from jax.experimental import pallas as pl
from jax.experimental.pallas import tpu as pltpu

_FILL = -10000.0
_VOCAB = 100000
_T = 2048
_RESTRICTED = 65
_VB = 1000
_N = _VOCAB // _VB
_RING = 8


def _copy(buf, out_ref, stripe, sem):
    return pltpu.make_async_copy(
        buf, out_ref.at[pl.ds(stripe * _VB, _VB)], sem)


def _body(hs_ref, w_ref, out_ref, fillbuf, buf0, sems, sem0):
    i = pl.program_id(0)

    @pl.when(i == 0)
    def _init_fill():
        fillbuf[...] = jnp.full((_VB, _T), _FILL, dtype=jnp.float32)

    @pl.when(i >= 1)
    def _stream():
        slot = lax.rem(i, _RING)

        @pl.when(i >= _RING + 1)
        def _reclaim():
            _copy(fillbuf, out_ref, i - _RING, sems.at[slot]).wait()

        _copy(fillbuf, out_ref, i, sems.at[slot]).start()

    @pl.when(i == 1)
    def _stripe0():
        buf0[...] = jnp.full((_VB, _T), _FILL, dtype=jnp.float32)
        logits_t = jax.lax.dot_general(
            w_ref[...], hs_ref[0],
            dimension_numbers=(((1,), (1,)), ((), ())),
            preferred_element_type=jnp.float32)
        buf0[100:164, :] = logits_t[0:64, :]
        buf0[999:1000, :] = logits_t[64:65, :]
        _copy(buf0, out_ref, 0, sem0).start()

    @pl.when(i == _N - 1)
    def _drain():
        for j in range(_RING):
            stripe = _N - _RING + j
            _copy(fillbuf, out_ref, stripe,
                  sems.at[lax.rem(stripe, _RING)]).wait()
        _copy(buf0, out_ref, 0, sem0).wait()


def kernel(hidden_states, W):
    B, T, H = hidden_states.shape
    hs = hidden_states.astype(jnp.float32)
    w_pad = jnp.zeros((128, H), dtype=jnp.float32).at[:_RESTRICTED].set(
        W.astype(jnp.float32))

    out_t = pl.pallas_call(
        _body,
        grid=(_N,),
        in_specs=[
            pl.BlockSpec((1, T, H), lambda i: (0, 0, 0)),
            pl.BlockSpec((128, H), lambda i: (0, 0)),
        ],
        out_specs=pl.BlockSpec(memory_space=pltpu.MemorySpace.HBM),
        out_shape=jax.ShapeDtypeStruct((_VOCAB, T), jnp.float32),
        scratch_shapes=[
            pltpu.VMEM((_VB, _T), jnp.float32),
            pltpu.VMEM((_VB, _T), jnp.float32),
            pltpu.SemaphoreType.DMA((_RING,)),
            pltpu.SemaphoreType.DMA,
        ],
    )(hs, w_pad)
    return jnp.swapaxes(out_t, 0, 1)[None]


# speedup vs baseline: 1.1041x; 1.0052x over previous
import jax
import jax.numpy as jnp
from jax import lax
from jax.experimental import pallas as pl
from jax.experimental.pallas import tpu as pltpu

_FILL = -10000.0
_VOCAB = 100000
_T = 2048
_RESTRICTED = 65
_VB = 1000
_N = _VOCAB // _VB
_RING = 8


def _copy(buf, out_ref, stripe, sem):
    return pltpu.make_async_copy(
        buf, out_ref.at[pl.ds(stripe * _VB, _VB)], sem)


def _body(hs_ref, w_ref, out_ref, fillbuf, buf0, sems, sem0):
    i = pl.program_id(0)

    @pl.when(i == 0)
    def _init_fill():
        fillbuf[...] = jnp.full((_VB, _T), _FILL, dtype=jnp.float32)

    @pl.when(i >= 1)
    def _stream():
        slot = lax.rem(i, _RING)

        @pl.when(i >= _RING + 1)
        def _reclaim():
            _copy(fillbuf, out_ref, i - _RING, sems.at[slot]).wait()

        _copy(fillbuf, out_ref, i, sems.at[slot]).start()

    @pl.when(i == 1)
    def _stripe0():
        buf0[...] = jnp.full((_VB, _T), _FILL, dtype=jnp.float32)
        logits_t = jax.lax.dot_general(
            w_ref[...], hs_ref[0],
            dimension_numbers=(((1,), (1,)), ((), ())),
            preferred_element_type=jnp.float32)
        buf0[100:164, :] = logits_t[0:64, :]
        buf0[999:1000, :] = logits_t[64:65, :]
        _copy(buf0, out_ref, 0, sem0).start()

    @pl.when(i == _N - 1)
    def _drain():
        for j in range(_RING):
            stripe = _N - _RING + j
            _copy(fillbuf, out_ref, stripe,
                  sems.at[lax.rem(stripe, _RING)]).wait()
        _copy(buf0, out_ref, 0, sem0).wait()


def kernel(hidden_states, W):
    B, T, H = hidden_states.shape
    hs = hidden_states.astype(jnp.float32)
    w = W.astype(jnp.float32)

    out_t = pl.pallas_call(
        _body,
        grid=(_N,),
        in_specs=[
            pl.BlockSpec((1, T, H), lambda i: (0, 0, 0)),
            pl.BlockSpec((_RESTRICTED, H), lambda i: (0, 0)),
        ],
        out_specs=pl.BlockSpec(memory_space=pltpu.MemorySpace.HBM),
        out_shape=jax.ShapeDtypeStruct((_VOCAB, T), jnp.float32),
        scratch_shapes=[
            pltpu.VMEM((_VB, _T), jnp.float32),
            pltpu.VMEM((_VB, _T), jnp.float32),
            pltpu.SemaphoreType.DMA((_RING,)),
            pltpu.SemaphoreType.DMA,
        ],
    )(hs, w)
    return jnp.swapaxes(out_t, 0, 1)[None]
